# baseline (speedup 1.0000x reference)
import jax
import jax.numpy as jnp
from jax.experimental import pallas as pl
from jax.experimental.pallas import tpu as pltpu

_TB = 1024
_K = 8192
_D = 256


def _cn_kernel(c_ref, cn_ref):
    c = c_ref[...]
    cn_ref[...] = jnp.sum(c * c, axis=1).reshape(1, _K)


def _assign_kernel(x_ref, c_ref, cn_ref, out_ref):
    xm2 = x_ref[...] * -2.0
    cn = cn_ref[...]
    rv, ri = None, None
    ch = 1024
    hw = ch // 2
    gbase = jax.lax.broadcasted_iota(
        jnp.int32, (_TB, hw), 1).astype(jnp.float32)
    for j in range(_K // ch):
        prod = jax.lax.dot_general(
            xm2, c_ref[j * ch:(j + 1) * ch, :], (((1,), (1,)), ((), ())),
            preferred_element_type=jnp.float32)
        pc = prod + cn[:, j * ch:(j + 1) * ch]
        v0, v1 = pc[:, :hw], pc[:, hw:]
        mask = v1 < v0
        cv = jnp.where(mask, v1, v0)
        g0 = gbase + float(j * ch)
        co = jnp.where(mask, g0 + float(hw), g0)
        for h in (256, 128):
            v0, v1 = cv[:, :h], cv[:, h:]
            o0, o1 = co[:, :h], co[:, h:]
            mask = v1 < v0
            cv = jnp.where(mask, v1, v0)
            co = jnp.where(mask, o1, o0)
        if rv is None:
            rv, ri = cv, co
        else:
            mask = cv < rv
            rv = jnp.where(mask, cv, rv)
            ri = jnp.where(mask, co, ri)
    m = jnp.min(rv, axis=1, keepdims=True)
    amin = jnp.min(jnp.where(rv == m, ri, float(_K)), axis=1)
    out_ref[...] = amin.astype(jnp.int32).reshape(_TB, 1)


def kernel(x, centers):
    b, t, d = x.shape
    n = b * t
    nblocks = n // _TB
    x2 = x.reshape(n, d)
    cn = pl.pallas_call(
        _cn_kernel,
        out_shape=jax.ShapeDtypeStruct((1, _K), jnp.float32),
    )(centers)
    out = pl.pallas_call(
        _assign_kernel,
        grid=(nblocks,),
        in_specs=[
            pl.BlockSpec((_TB, _D), lambda i: (i, 0)),
            pl.BlockSpec((_K, _D), lambda i: (0, 0)),
            pl.BlockSpec((1, _K), lambda i: (0, 0)),
        ],
        out_specs=pl.BlockSpec((_TB, 1), lambda i: (i, 0)),
        out_shape=jax.ShapeDtypeStruct((n, 1), jnp.int32),
        compiler_params=pltpu.CompilerParams(
            dimension_semantics=("parallel",),
        ),
    )(x2, centers, cn)
    return out.reshape(b, t)


# speedup vs baseline: 1.0336x; 1.0336x over previous
import jax
import jax.numpy as jnp
from jax.experimental import pallas as pl
from jax.experimental.pallas import tpu as pltpu

_TB = 1024
_K = 8192
_D = 256


def _assign_kernel(x_ref, c_ref, out_ref, cn_ref):
    @pl.when(pl.program_id(0) == 0)
    def _():
        c = c_ref[...]
        cn_ref[...] = jnp.sum(c * c, axis=1).reshape(1, _K)

    xm2 = x_ref[...] * -2.0
    cn = cn_ref[...]
    rv, ri = None, None
    ch = 1024
    hw = ch // 2
    gbase = jax.lax.broadcasted_iota(
        jnp.int32, (_TB, hw), 1).astype(jnp.float32)
    for j in range(_K // ch):
        prod = jax.lax.dot_general(
            xm2, c_ref[j * ch:(j + 1) * ch, :], (((1,), (1,)), ((), ())),
            preferred_element_type=jnp.float32)
        pc = prod + cn[:, j * ch:(j + 1) * ch]
        v0, v1 = pc[:, :hw], pc[:, hw:]
        mask = v1 < v0
        cv = jnp.where(mask, v1, v0)
        g0 = gbase + float(j * ch)
        co = jnp.where(mask, g0 + float(hw), g0)
        for h in (256, 128):
            v0, v1 = cv[:, :h], cv[:, h:]
            o0, o1 = co[:, :h], co[:, h:]
            mask = v1 < v0
            cv = jnp.where(mask, v1, v0)
            co = jnp.where(mask, o1, o0)
        if rv is None:
            rv, ri = cv, co
        else:
            mask = cv < rv
            rv = jnp.where(mask, cv, rv)
            ri = jnp.where(mask, co, ri)
    m = jnp.min(rv, axis=1, keepdims=True)
    amin = jnp.min(jnp.where(rv == m, ri, float(_K)), axis=1)
    out_ref[...] = amin.astype(jnp.int32).reshape(_TB, 1)


def kernel(x, centers):
    b, t, d = x.shape
    n = b * t
    nblocks = n // _TB
    x2 = x.reshape(n, d)
    out = pl.pallas_call(
        _assign_kernel,
        grid=(nblocks,),
        in_specs=[
            pl.BlockSpec((_TB, _D), lambda i: (i, 0)),
            pl.BlockSpec((_K, _D), lambda i: (0, 0)),
        ],
        out_specs=pl.BlockSpec((_TB, 1), lambda i: (i, 0)),
        out_shape=jax.ShapeDtypeStruct((n, 1), jnp.int32),
        scratch_shapes=[pltpu.VMEM((1, _K), jnp.float32)],
        compiler_params=pltpu.CompilerParams(
            dimension_semantics=("arbitrary",),
        ),
    )(x2, centers)
    return out.reshape(b, t)
